# baseline (speedup 1.0000x reference)
import functools

import jax
import jax.numpy as jnp
from jax import lax
from jax.experimental import pallas as pl
from jax.experimental.pallas import tpu as pltpu
from jax.experimental.pallas import tpu_sc as plsc

EMBED = 128
BLOCKS = 5
HIDDEN = 32

NC = 2
NS = 16
NW = NC * NS
CHUNK = 64
GC = 32


def _sc_mesh():
    return plsc.VectorSubcoreMesh(core_axis_name="c", subcore_axis_name="s")


def _make_count_kernel(n_pad, nch):
    rows_per = n_pad // NS

    @functools.partial(
        pl.kernel,
        out_type=jax.ShapeDtypeStruct((NC, n_pad, EMBED), jnp.float32),
        mesh=_sc_mesh(),
        scratch_types=[
            pltpu.VMEM((nch, CHUNK), jnp.int32),
            pltpu.VMEM((CHUNK, EMBED), jnp.float32),
            pltpu.VMEM_SHARED((n_pad, EMBED), jnp.float32),
            pltpu.SemaphoreType.DMA,
        ],
    )
    def count_kernel(col_hbm, ones_hbm, zeros_hbm, out_hbm, colv, onesv, accum, sem):
        c = lax.axis_index("c")
        s = lax.axis_index("s")
        wid = s * NC + c
        pltpu.sync_copy(zeros_hbm.at[pl.ds(s * rows_per, rows_per)],
                        accum.at[pl.ds(s * rows_per, rows_per)])
        pltpu.sync_copy(col_hbm.at[wid], colv)
        pltpu.sync_copy(ones_hbm, onesv)
        plsc.subcore_barrier()

        def body(j, carry):
            pltpu.sync_copy(onesv, accum.at[colv.at[j]], add=True)
            return carry

        lax.fori_loop(0, nch, body, 0)
        plsc.subcore_barrier()
        pltpu.sync_copy(accum.at[pl.ds(s * rows_per, rows_per)],
                        out_hbm.at[c, pl.ds(s * rows_per, rows_per)])

    return count_kernel


NBUF = 5


def _make_agg_kernel(n_pad, ng):
    rows_per = n_pad // NS

    @functools.partial(
        pl.kernel,
        out_type=jax.ShapeDtypeStruct((NC, n_pad, EMBED), jnp.float32),
        mesh=_sc_mesh(),
        scratch_types=[
            pltpu.VMEM((GC, CHUNK), jnp.int32),
            pltpu.VMEM((GC, CHUNK), jnp.int32),
        ] + [pltpu.VMEM((CHUNK, EMBED), jnp.float32) for _ in range(NBUF)] + [
            pltpu.VMEM_SHARED((n_pad, EMBED), jnp.float32),
        ] + [pltpu.SemaphoreType.DMA for _ in range(NBUF)],
    )
    def agg_kernel(hs_hbm, row_hbm, col_hbm, zeros_hbm, out_hbm,
                   rowv, colv, *rest):
        gb = rest[:NBUF]
        accum = rest[NBUF]
        sem = rest[NBUF + 1:]
        c = lax.axis_index("c")
        s = lax.axis_index("s")
        wid = s * NC + c
        pltpu.sync_copy(zeros_hbm.at[pl.ds(s * rows_per, rows_per)],
                        accum.at[pl.ds(s * rows_per, rows_per)])
        plsc.subcore_barrier()

        def group(g, carry):
            pltpu.sync_copy(row_hbm.at[wid, g], rowv)
            pltpu.sync_copy(col_hbm.at[wid, g], colv)
            for b in range(NBUF):
                pltpu.async_copy(hs_hbm.at[rowv.at[b]], gb[b], sem[b])

            def body(j, carry2):
                for b in range(NBUF):
                    @pl.when(j % NBUF == b)
                    def _(b=b):
                        pltpu.make_async_copy(hs_hbm.at[rowv.at[j]],
                                              gb[b], sem[b]).wait()
                        pltpu.sync_copy(gb[b], accum.at[colv.at[j]], add=True)

                        @pl.when(j + NBUF < GC)
                        def _():
                            pltpu.async_copy(hs_hbm.at[rowv.at[j + NBUF]],
                                             gb[b], sem[b])
                return carry2

            lax.fori_loop(0, GC, body, 0)
            return carry

        lax.fori_loop(0, ng, group, 0)
        plsc.subcore_barrier()
        pltpu.sync_copy(accum.at[pl.ds(s * rows_per, rows_per)],
                        out_hbm.at[c, pl.ds(s * rows_per, rows_per)])

    return agg_kernel


BR = 2048


def _embed_body(x_ref, counts_ref, wet_ref, be_ref, h_ref, hs_ref, deg_ref):
    x = x_ref[...]
    cnt = counts_ref[0, :, 0:1] + counts_ref[1, :, 0:1]
    deg = lax.rsqrt(cnt + 1.0)
    h = jnp.maximum(jnp.dot(x, wet_ref[...],
                            preferred_element_type=jnp.float32) + be_ref[...], 0.0)
    h_ref[...] = h
    hs_ref[...] = deg * h
    deg_ref[...] = deg


def _gru_body(h_ref, p_ref, deg_ref, maxh_ref, wih_ref, whh_ref, bih_ref,
              bhh_ref, h_out, hs_out, maxh_out):
    h = h_ref[...]
    deg = deg_ref[...]
    aggr = deg * (p_ref[0] + p_ref[1])
    gi = jnp.dot(h, wih_ref[...], preferred_element_type=jnp.float32) + bih_ref[...]
    gh = jnp.dot(aggr, whh_ref[...], preferred_element_type=jnp.float32) + bhh_ref[...]
    r = jax.nn.sigmoid(gi[:, :EMBED] + gh[:, :EMBED])
    z = jax.nn.sigmoid(gi[:, EMBED:2 * EMBED] + gh[:, EMBED:2 * EMBED])
    n = jnp.tanh(gi[:, 2 * EMBED:] + r * gh[:, 2 * EMBED:])
    hn = (1.0 - z) * n + z * aggr
    h_out[...] = hn
    hs_out[...] = deg * hn
    maxh_out[...] = jnp.maximum(maxh_ref[...], hn)


def _dec_body(maxh_ref, wdh_ref, bdh_ref, wdo_ref, bdo_ref, out_ref):
    zz = jnp.maximum(jnp.dot(maxh_ref[...], wdh_ref[...],
                             preferred_element_type=jnp.float32) + bdh_ref[...], 0.0)
    out_ref[...] = jnp.dot(zz, wdo_ref[...],
                           preferred_element_type=jnp.float32) + bdo_ref[...]


def _row_spec(width):
    return pl.BlockSpec((BR, width), lambda i: (i, 0))


def _full_spec(shape):
    nd = len(shape)
    return pl.BlockSpec(shape, lambda i, _n=nd: (0,) * _n)


def kernel(x, edge_idx, W_embed, b_embed, W_ih, W_hh, b_ih, b_hh,
           W_dh, b_dh, W_do, b_do):
    N = x.shape[0]
    E = edge_idx.shape[1]
    n_pad = ((N + BR - 1) // BR) * BR
    group_edges = NW * GC * CHUNK
    ng = (E + group_edges - 1) // group_edges
    nch = ng * GC
    e_pad = ng * group_edges

    row = edge_idx[0]
    col = edge_idx[1]
    pad_idx = jnp.full((e_pad - E,), N, dtype=jnp.int32)
    row4 = jnp.concatenate([row, pad_idx]).reshape(NW, ng, GC, CHUNK)
    col4 = jnp.concatenate([col, pad_idx]).reshape(NW, ng, GC, CHUNK)
    col3 = col4.reshape(NW, nch, CHUNK)

    x_pad = jnp.zeros((n_pad, 3), jnp.float32).at[:N].set(x)
    zeros_nd = jnp.zeros((n_pad, EMBED), jnp.float32)
    ones_w = jnp.ones((CHUNK, EMBED), jnp.float32)

    wet = W_embed.T
    be = b_embed.reshape(1, EMBED)
    wih = W_ih.T
    whh = W_hh.T
    bih = b_ih.reshape(1, 3 * EMBED)
    bhh = b_hh.reshape(1, 3 * EMBED)
    wdh = W_dh.T
    bdh = b_dh.reshape(1, HIDDEN)
    wdo = W_do.T
    bdo = b_do.reshape(1, 1)

    counts = _make_count_kernel(n_pad, nch)(col3, ones_w, zeros_nd)

    grid = n_pad // BR
    h, hs, deg = pl.pallas_call(
        _embed_body,
        grid=(grid,),
        in_specs=[
            _row_spec(3),
            pl.BlockSpec((NC, BR, EMBED), lambda i: (0, i, 0)),
            _full_spec((3, EMBED)),
            _full_spec((1, EMBED)),
        ],
        out_specs=[_row_spec(EMBED), _row_spec(EMBED), _row_spec(1)],
        out_shape=[
            jax.ShapeDtypeStruct((n_pad, EMBED), jnp.float32),
            jax.ShapeDtypeStruct((n_pad, EMBED), jnp.float32),
            jax.ShapeDtypeStruct((n_pad, 1), jnp.float32),
        ],
    )(x_pad, counts, wet, be)

    agg = _make_agg_kernel(n_pad, ng)
    gru = pl.pallas_call(
        _gru_body,
        grid=(grid,),
        in_specs=[
            _row_spec(EMBED),
            pl.BlockSpec((NC, BR, EMBED), lambda i: (0, i, 0)),
            _row_spec(1),
            _row_spec(EMBED),
            _full_spec((EMBED, 3 * EMBED)),
            _full_spec((EMBED, 3 * EMBED)),
            _full_spec((1, 3 * EMBED)),
            _full_spec((1, 3 * EMBED)),
        ],
        out_specs=[_row_spec(EMBED), _row_spec(EMBED), _row_spec(EMBED)],
        out_shape=[
            jax.ShapeDtypeStruct((n_pad, EMBED), jnp.float32),
            jax.ShapeDtypeStruct((n_pad, EMBED), jnp.float32),
            jax.ShapeDtypeStruct((n_pad, EMBED), jnp.float32),
        ],
    )

    maxh = h
    for _ in range(BLOCKS):
        partials = agg(hs, row4, col4, zeros_nd)
        h, hs, maxh = gru(h, partials, deg, maxh, wih, whh, bih, bhh)

    out = pl.pallas_call(
        _dec_body,
        grid=(grid,),
        in_specs=[
            _row_spec(EMBED),
            _full_spec((EMBED, HIDDEN)),
            _full_spec((1, HIDDEN)),
            _full_spec((HIDDEN, 1)),
            _full_spec((1, 1)),
        ],
        out_specs=_row_spec(1),
        out_shape=jax.ShapeDtypeStruct((n_pad, 1), jnp.float32),
    )(maxh, wdh, bdh, wdo, bdo)

    return out[:N]


# speedup vs baseline: 1.4316x; 1.4316x over previous
import functools

import jax
import jax.numpy as jnp
from jax import lax
from jax.experimental import pallas as pl
from jax.experimental.pallas import tpu as pltpu
from jax.experimental.pallas import tpu_sc as plsc

EMBED = 128
BLOCKS = 5
HIDDEN = 32

NC = 2
NS = 16
NW = NC * NS
CHUNK = 128
GC = 32
NG0 = 8
NG1 = 2


def _sc_mesh():
    return plsc.VectorSubcoreMesh(core_axis_name="c", subcore_axis_name="s")


def _make_count_kernel(n_pad, nch):
    rows_per = n_pad // NS

    @functools.partial(
        pl.kernel,
        out_type=jax.ShapeDtypeStruct((NC, n_pad, EMBED), jnp.float32),
        mesh=_sc_mesh(),
        scratch_types=[
            pltpu.VMEM((nch, CHUNK), jnp.int32),
            pltpu.VMEM((CHUNK, EMBED), jnp.float32),
            pltpu.VMEM_SHARED((n_pad, EMBED), jnp.float32),
            pltpu.SemaphoreType.DMA,
        ],
    )
    def count_kernel(col_hbm, ones_hbm, zeros_hbm, out_hbm, colv, onesv, accum, sem):
        c = lax.axis_index("c")
        s = lax.axis_index("s")
        wid = s * NC + c
        pltpu.sync_copy(zeros_hbm.at[pl.ds(s * rows_per, rows_per)],
                        accum.at[pl.ds(s * rows_per, rows_per)])
        pltpu.sync_copy(col_hbm.at[wid], colv)
        pltpu.sync_copy(ones_hbm, onesv)
        plsc.subcore_barrier()

        def body(j, carry):
            pltpu.sync_copy(onesv, accum.at[colv.at[j]], add=True)
            return carry

        lax.fori_loop(0, nch, body, 0)
        plsc.subcore_barrier()
        pltpu.sync_copy(accum.at[pl.ds(s * rows_per, rows_per)],
                        out_hbm.at[c, pl.ds(s * rows_per, rows_per)])

    return count_kernel


NBUF = 2


def _make_agg_kernel(n_pad):
    rows_per = n_pad // NS

    @functools.partial(
        pl.kernel,
        out_type=jax.ShapeDtypeStruct((NC, n_pad, EMBED), jnp.float32),
        mesh=_sc_mesh(),
        scratch_types=[
            pltpu.VMEM((GC, CHUNK), jnp.int32),
            pltpu.VMEM((GC, CHUNK), jnp.int32),
        ] + [pltpu.VMEM((CHUNK, EMBED), jnp.float32) for _ in range(NBUF)] + [
            pltpu.VMEM_SHARED((n_pad, EMBED), jnp.float32),
        ] + [pltpu.SemaphoreType.DMA for _ in range(NBUF)],
    )
    def agg_kernel(hs_hbm, row_hbm, col_hbm, zeros_hbm, out_hbm,
                   rowv, colv, *rest):
        gb = rest[:NBUF]
        accum = rest[NBUF]
        sem = rest[NBUF + 1:]
        c = lax.axis_index("c")
        s = lax.axis_index("s")
        base = jnp.where(c == 0, s * NG0, NS * NG0 + s * NG1)
        ng_c = jnp.where(c == 0, NG0, NG1)
        pltpu.sync_copy(zeros_hbm.at[pl.ds(s * rows_per, rows_per)],
                        accum.at[pl.ds(s * rows_per, rows_per)])
        plsc.subcore_barrier()

        def group(g, carry):
            pltpu.sync_copy(row_hbm.at[base + g], rowv)
            pltpu.sync_copy(col_hbm.at[base + g], colv)
            for b in range(NBUF):
                pltpu.async_copy(hs_hbm.at[rowv.at[b]], gb[b], sem[b])

            def body(j, carry2):
                for b in range(NBUF):
                    @pl.when(j % NBUF == b)
                    def _(b=b):
                        pltpu.make_async_copy(hs_hbm.at[rowv.at[j]],
                                              gb[b], sem[b]).wait()
                        pltpu.sync_copy(gb[b], accum.at[colv.at[j]], add=True)

                        @pl.when(j + NBUF < GC)
                        def _():
                            pltpu.async_copy(hs_hbm.at[rowv.at[j + NBUF]],
                                             gb[b], sem[b])
                return carry2

            lax.fori_loop(0, GC, body, 0)
            return carry

        lax.fori_loop(0, ng_c, group, 0)
        plsc.subcore_barrier()
        pltpu.sync_copy(accum.at[pl.ds(s * rows_per, rows_per)],
                        out_hbm.at[c, pl.ds(s * rows_per, rows_per)])

    return agg_kernel


BR = 2048


def _embed_body(x_ref, counts_ref, wet_ref, be_ref, h_ref, hs_ref, deg_ref):
    x = x_ref[...]
    cnt = counts_ref[0, :, 0:1] + counts_ref[1, :, 0:1]
    deg = lax.rsqrt(cnt + 1.0)
    h = jnp.maximum(jnp.dot(x, wet_ref[...],
                            preferred_element_type=jnp.float32) + be_ref[...], 0.0)
    h_ref[...] = h
    hs_ref[...] = deg * h
    deg_ref[...] = deg


def _gru_body(h_ref, p_ref, deg_ref, maxh_ref, wih_ref, whh_ref, bih_ref,
              bhh_ref, h_out, hs_out, maxh_out):
    h = h_ref[...]
    deg = deg_ref[...]
    aggr = deg * (p_ref[0] + p_ref[1])
    gi = jnp.dot(h, wih_ref[...], preferred_element_type=jnp.float32) + bih_ref[...]
    gh = jnp.dot(aggr, whh_ref[...], preferred_element_type=jnp.float32) + bhh_ref[...]
    r = jax.nn.sigmoid(gi[:, :EMBED] + gh[:, :EMBED])
    z = jax.nn.sigmoid(gi[:, EMBED:2 * EMBED] + gh[:, EMBED:2 * EMBED])
    n = jnp.tanh(gi[:, 2 * EMBED:] + r * gh[:, 2 * EMBED:])
    hn = (1.0 - z) * n + z * aggr
    h_out[...] = hn
    hs_out[...] = deg * hn
    maxh_out[...] = jnp.maximum(maxh_ref[...], hn)


def _dec_body(maxh_ref, wdh_ref, bdh_ref, wdo_ref, bdo_ref, out_ref):
    zz = jnp.maximum(jnp.dot(maxh_ref[...], wdh_ref[...],
                             preferred_element_type=jnp.float32) + bdh_ref[...], 0.0)
    out_ref[...] = jnp.dot(zz, wdo_ref[...],
                           preferred_element_type=jnp.float32) + bdo_ref[...]


def _row_spec(width):
    return pl.BlockSpec((BR, width), lambda i: (i, 0))


def _full_spec(shape):
    nd = len(shape)
    return pl.BlockSpec(shape, lambda i, _n=nd: (0,) * _n)


def kernel(x, edge_idx, W_embed, b_embed, W_ih, W_hh, b_ih, b_hh,
           W_dh, b_dh, W_do, b_do):
    N = x.shape[0]
    E = edge_idx.shape[1]
    n_pad = ((N + BR - 1) // BR) * BR
    tot_groups = NS * (NG0 + NG1)
    e_pad = tot_groups * GC * CHUNK
    nch = e_pad // (NW * CHUNK)

    row = edge_idx[0]
    col = edge_idx[1]
    pad_idx = jnp.full((e_pad - E,), N, dtype=jnp.int32)
    row_g = jnp.concatenate([row, pad_idx]).reshape(tot_groups, GC, CHUNK)
    col_g = jnp.concatenate([col, pad_idx]).reshape(tot_groups, GC, CHUNK)
    col3 = col_g.reshape(NW, nch, CHUNK)

    x_pad = jnp.zeros((n_pad, 3), jnp.float32).at[:N].set(x)
    zeros_nd = jnp.zeros((n_pad, EMBED), jnp.float32)
    ones_w = jnp.ones((CHUNK, EMBED), jnp.float32)

    wet = W_embed.T
    be = b_embed.reshape(1, EMBED)
    wih = W_ih.T
    whh = W_hh.T
    bih = b_ih.reshape(1, 3 * EMBED)
    bhh = b_hh.reshape(1, 3 * EMBED)
    wdh = W_dh.T
    bdh = b_dh.reshape(1, HIDDEN)
    wdo = W_do.T
    bdo = b_do.reshape(1, 1)

    counts = _make_count_kernel(n_pad, nch)(col3, ones_w, zeros_nd)

    grid = n_pad // BR
    h, hs, deg = pl.pallas_call(
        _embed_body,
        grid=(grid,),
        in_specs=[
            _row_spec(3),
            pl.BlockSpec((NC, BR, EMBED), lambda i: (0, i, 0)),
            _full_spec((3, EMBED)),
            _full_spec((1, EMBED)),
        ],
        out_specs=[_row_spec(EMBED), _row_spec(EMBED), _row_spec(1)],
        out_shape=[
            jax.ShapeDtypeStruct((n_pad, EMBED), jnp.float32),
            jax.ShapeDtypeStruct((n_pad, EMBED), jnp.float32),
            jax.ShapeDtypeStruct((n_pad, 1), jnp.float32),
        ],
    )(x_pad, counts, wet, be)

    agg = _make_agg_kernel(n_pad)
    gru = pl.pallas_call(
        _gru_body,
        grid=(grid,),
        in_specs=[
            _row_spec(EMBED),
            pl.BlockSpec((NC, BR, EMBED), lambda i: (0, i, 0)),
            _row_spec(1),
            _row_spec(EMBED),
            _full_spec((EMBED, 3 * EMBED)),
            _full_spec((EMBED, 3 * EMBED)),
            _full_spec((1, 3 * EMBED)),
            _full_spec((1, 3 * EMBED)),
        ],
        out_specs=[_row_spec(EMBED), _row_spec(EMBED), _row_spec(EMBED)],
        out_shape=[
            jax.ShapeDtypeStruct((n_pad, EMBED), jnp.float32),
            jax.ShapeDtypeStruct((n_pad, EMBED), jnp.float32),
            jax.ShapeDtypeStruct((n_pad, EMBED), jnp.float32),
        ],
    )

    maxh = h
    for _ in range(BLOCKS):
        partials = agg(hs, row_g, col_g, zeros_nd)
        h, hs, maxh = gru(h, partials, deg, maxh, wih, whh, bih, bhh)

    out = pl.pallas_call(
        _dec_body,
        grid=(grid,),
        in_specs=[
            _row_spec(EMBED),
            _full_spec((EMBED, HIDDEN)),
            _full_spec((1, HIDDEN)),
            _full_spec((HIDDEN, 1)),
            _full_spec((1, 1)),
        ],
        out_specs=_row_spec(1),
        out_shape=jax.ShapeDtypeStruct((n_pad, 1), jnp.float32),
    )(maxh, wdh, bdh, wdo, bdo)

    return out[:N]
